# baseline (speedup 1.0000x reference)
import functools

import jax
import jax.numpy as jnp
from jax import lax
from jax.experimental import pallas as pl
from jax.experimental.pallas import tpu as pltpu
from jax.experimental.pallas import tpu_sc as plsc

N0, N1, N2 = 50000, 10000, 2048
HEADS, HID = 4, 32
FEAT = HEADS * HID
ROW = 144
NEG_SLOPE = 0.2
W = 256
NS = 16
NC = 2


def _proj_body(x_ref, W_ref, b_ref, SW_ref, Sb_ref, z_ref, s_ref):
    xb = x_ref[...]
    z_ref[...] = (
        jnp.dot(xb, W_ref[...], preferred_element_type=jnp.float32) + b_ref[...]
    )
    s_ref[...] = (
        jnp.dot(xb, SW_ref[...], preferred_element_type=jnp.float32) + Sb_ref[...]
    )


def _proj(x, Wm, b, SW, Sb, block_rows):
    n, k = x.shape
    m1 = Wm.shape[1]
    m2 = SW.shape[1]
    grid = n // block_rows
    return pl.pallas_call(
        _proj_body,
        grid=(grid,),
        in_specs=[
            pl.BlockSpec((block_rows, k), lambda i: (i, 0)),
            pl.BlockSpec((k, m1), lambda i: (0, 0)),
            pl.BlockSpec((1, m1), lambda i: (0, 0)),
            pl.BlockSpec((k, m2), lambda i: (0, 0)),
            pl.BlockSpec((1, m2), lambda i: (0, 0)),
        ],
        out_specs=[
            pl.BlockSpec((block_rows, m1), lambda i: (i, 0)),
            pl.BlockSpec((block_rows, m2), lambda i: (i, 0)),
        ],
        out_shape=[
            jax.ShapeDtypeStruct((n, m1), jnp.float32),
            jax.ShapeDtypeStruct((n, m2), jnp.float32),
        ],
    )(x, Wm, b.reshape(1, -1), SW, Sb.reshape(1, -1))


def _chunks(total, maxc):
    out = []
    off = 0
    while off < total:
        c = min(maxc, total - off)
        out.append((off, c))
        off += c
    return out


def _make_edge_kernel(E, ND, ZR, R):
    ECR = E // NS
    NWIN = ECR // W
    assert ECR % W == 0 and ND % (NC * R * 128) == 0
    QND = ND // (NC * R)
    NDL = QND + 128
    RPTZ = NDL // NS
    RPTO = QND // NS
    mesh = plsc.VectorSubcoreMesh(core_axis_name="c", subcore_axis_name="s")

    RW = 512
    T = ECR // (2 * RW)
    PCAP = 768
    SCW = ECR + 2 * W
    assert ECR % (2 * RW) == 0

    def _body(z_hbm, src_hbm, dst_hbm, att_hbm, out_hbm, scr_hbm,
              raws, rawd, pends, pendg, pendd, procs, procg, procd,
              zsrc, zdst, upd, attv, acc,
              sem1, sem2, sas, sad, sbs, sbd):
        cid = lax.axis_index("c")
        sid = lax.axis_index("s")
        wid = cid * NS + sid
        lanes = lax.broadcasted_iota(jnp.int32, (16,), 0)
        zero16 = jnp.zeros((16,), jnp.float32)

        pltpu.sync_copy(att_hbm, attv)

        atts = [attv[pl.ds(j * 16, 16)] for j in range(8)]

        def ebody(kk, c):
            for u in range(4):
                k = kk * 4 + u
                zs_list = []
                exs = []
                for h in range(HEADS):
                    part = None
                    for j in (2 * h, 2 * h + 1):
                        zs = zsrc[k, pl.ds(j * 16, 16)]
                        zd = zdst[k, pl.ds(j * 16, 16)]
                        zs_list.append(zs)
                        e = zs + zd
                        m = jnp.maximum(e, NEG_SLOPE * e) * atts[j]
                        part = m if part is None else part + m
                    s = jnp.sum(part)
                    exs.append(jnp.exp(lax.broadcast(s, (16,))))
                for h in range(HEADS):
                    for jj in range(2):
                        j = 2 * h + jj
                        upd[k, pl.ds(j * 16, 16)] = zs_list[j] * exs[h]
                tail = jnp.where(
                    lanes == 0, exs[0],
                    jnp.where(lanes == 1, exs[1],
                              jnp.where(lanes == 2, exs[2],
                                        jnp.where(lanes == 3, exs[3],
                                                  zero16))))
                upd[k, pl.ds(FEAT, 16)] = tail
            return c

        def dump_window(nw):
            pltpu.sync_copy(pends.at[pl.ds(0, W)],
                            scr_hbm.at[wid, 0, pl.ds(nw * W, W)])
            pltpu.sync_copy(pendg.at[pl.ds(0, W)],
                            scr_hbm.at[wid, 1, pl.ds(nw * W, W)])
            pltpu.sync_copy(pendd.at[pl.ds(0, W)],
                            scr_hbm.at[wid, 2, pl.ds(nw * W, W)])
            for j in range((PCAP - W) // 16):
                pends[pl.ds(j * 16, 16)] = pends[pl.ds(W + j * 16, 16)]
                pendg[pl.ds(j * 16, 16)] = pendg[pl.ds(W + j * 16, 16)]
                pendd[pl.ds(j * 16, 16)] = pendd[pl.ds(W + j * 16, 16)]

        t0 = sid * ECR

        def issue_raw(w, rb, ss, sd):
            c1 = pltpu.async_copy(src_hbm.at[pl.ds(t0 + w * RW, RW)],
                                  raws.at[rb], ss)
            c2 = pltpu.async_copy(dst_hbm.at[pl.ds(t0 + w * RW, RW)],
                                  rawd.at[rb], sd)
            del c1, c2

        def wait_raw(rb, ss, sd):
            pltpu.make_async_copy(src_hbm.at[pl.ds(0, RW)],
                                  raws.at[rb], ss).wait()
            pltpu.make_async_copy(dst_hbm.at[pl.ds(0, RW)],
                                  rawd.at[rb], sd).wait()

        def rbody(r, _rc):
            lo = (cid * R + r) * QND
            lov = lax.broadcast(lo, (16,))
            hiv = lax.broadcast(lo + QND, (16,))

            def zbody(rr, c):
                for u in range(4):
                    for j in range(ROW // 16):
                        upd[rr * 4 + u, pl.ds(j * 16, 16)] = zero16
                return c

            lax.fori_loop(0, W // 4, zbody, 0)
            zbase = sid * RPTZ
            for off, c in _chunks(RPTZ, W):
                pltpu.sync_copy(upd.at[pl.ds(0, c)],
                                acc.at[pl.ds(zbase + off, c)])
            plsc.subcore_barrier()

            def compact_half(rb, g0, state):
                ptr, nw = state
                for j in range(g0, g0 + W // 16):
                    s = raws[rb, pl.ds(j * 16, 16)]
                    d = rawd[rb, pl.ds(j * 16, 16)]
                    msk = (d >= lov) & (d < hiv)
                    mi = msk.astype(jnp.int32)
                    pos = plsc.cumsum(mi) + lax.broadcast(ptr, (16,)) - 1
                    plsc.store_scatter(pends, [pos], s, mask=msk)
                    plsc.store_scatter(pendg, [pos], d, mask=msk)
                    plsc.store_scatter(pendd, [pos], d - lov, mask=msk)
                    ptr = ptr + jnp.sum(mi)

                @pl.when(ptr >= W)
                def _():
                    dump_window(nw)

                full = ptr >= W
                return (jnp.where(full, ptr - W, ptr),
                        jnp.where(full, nw + 1, nw))

            issue_raw(0, 0, sas, sad)

            def tbody(t, state):
                issue_raw(2 * t + 1, 1, sbs, sbd)
                wait_raw(0, sas, sad)
                state = compact_half(0, 0, state)
                state = compact_half(0, W // 16, state)

                @pl.when(t + 1 < T)
                def _():
                    issue_raw(2 * t + 2, 0, sas, sad)

                wait_raw(1, sbs, sbd)
                state = compact_half(1, 0, state)
                state = compact_half(1, W // 16, state)
                return state

            ptr, nw = lax.fori_loop(0, T, tbody,
                                    (jnp.int32(0), jnp.int32(0)))

            for j in range(PCAP // 16):
                gi = lanes + j * 16
                pad = gi >= lax.broadcast(ptr, (16,))
                tsrc = gi % ZR
                tdst = QND + (gi % 128)
                sv = pends[pl.ds(j * 16, 16)]
                pends[pl.ds(j * 16, 16)] = jnp.where(pad, tsrc, sv)
                gv = pendg[pl.ds(j * 16, 16)]
                pendg[pl.ds(j * 16, 16)] = jnp.where(pad, tsrc, gv)
                dv = pendd[pl.ds(j * 16, 16)]
                pendd[pl.ds(j * 16, 16)] = jnp.where(pad, tdst, dv)
            dump_window(nw)
            dump_window(nw + 1)
            nw = nw + 2

            def pbody(w, c):
                pltpu.sync_copy(scr_hbm.at[wid, 0, pl.ds(w * W, W)],
                                procs.at[0])
                pltpu.sync_copy(scr_hbm.at[wid, 1, pl.ds(w * W, W)],
                                procg.at[0])
                pltpu.sync_copy(scr_hbm.at[wid, 2, pl.ds(w * W, W)],
                                procd.at[0])
                cp1 = pltpu.async_copy(z_hbm.at[procs.at[0]], zsrc, sem1)
                cp2 = pltpu.async_copy(z_hbm.at[procg.at[0]], zdst, sem2)
                cp1.wait()
                cp2.wait()
                lax.fori_loop(0, W // 4, ebody, 0)
                pltpu.sync_copy(upd, acc.at[procd.at[0]], add=True)
                return c

            lax.fori_loop(0, nw, pbody, 0)

            plsc.subcore_barrier()
            obase = sid * RPTO
            for off, c in _chunks(RPTO, W):
                pltpu.sync_copy(acc.at[pl.ds(obase + off, c)],
                                out_hbm.at[cid * R + r, pl.ds(obase + off, c)])
            plsc.subcore_barrier()
            return _rc

        lax.fori_loop(0, R, rbody, 0)

    @functools.partial(
        pl.kernel,
        mesh=mesh,
        compiler_params=pltpu.CompilerParams(
            needs_layout_passes=False, use_tc_tiling_on_sc=False),
        out_type=(
            jax.ShapeDtypeStruct((NC * R, QND, ROW), jnp.float32),
            jax.ShapeDtypeStruct((NC * NS, 3, SCW), jnp.int32),
        ),
        scratch_types=[
            pltpu.VMEM((2, RW), jnp.int32),
            pltpu.VMEM((2, RW), jnp.int32),
            pltpu.VMEM((PCAP,), jnp.int32),
            pltpu.VMEM((PCAP,), jnp.int32),
            pltpu.VMEM((PCAP,), jnp.int32),
            pltpu.VMEM((1, W), jnp.int32),
            pltpu.VMEM((1, W), jnp.int32),
            pltpu.VMEM((1, W), jnp.int32),
            pltpu.VMEM((W, FEAT), jnp.float32),
            pltpu.VMEM((W, FEAT), jnp.float32),
            pltpu.VMEM((W, ROW), jnp.float32),
            pltpu.VMEM((FEAT,), jnp.float32),
            pltpu.VMEM_SHARED((NDL, ROW), jnp.float32),
            pltpu.SemaphoreType.DMA,
            pltpu.SemaphoreType.DMA,
            pltpu.SemaphoreType.DMA,
            pltpu.SemaphoreType.DMA,
            pltpu.SemaphoreType.DMA,
            pltpu.SemaphoreType.DMA,
        ],
    )
    def edge_kernel(z_hbm, src_hbm, dst_hbm, att_hbm, out_hbm, scr_hbm,
                    *scratch):
        _body(z_hbm, src_hbm, dst_hbm, att_hbm, out_hbm, scr_hbm, *scratch)

    def run(z, src, dst, att):
        return edge_kernel(z, src, dst, att)[0]

    return run


E0, E0P = 320000, 327680
E1 = 65536
NDP0 = 10240
_edge0 = _make_edge_kernel(E0P, NDP0, N1, 4)
_edge1 = _make_edge_kernel(E1, N2, N2, 2)


def _comb0_body(acc_ref, s0_ref, b_ref, h_ref):
    a = acc_ref[...]
    parts = []
    for h in range(HEADS):
        m = a[:, h * HID:(h + 1) * HID]
        d = a[:, FEAT + h:FEAT + h + 1]
        parts.append(m / (d + 1e-16))
    o = jnp.concatenate(parts, axis=1) + b_ref[...] + s0_ref[...]
    h_ref[...] = jnp.where(o > 0, o, jnp.exp(jnp.minimum(o, 0.0)) - 1.0)


def _comb0(acc, s0, bias0, block_rows):
    n = s0.shape[0]
    grid = n // block_rows
    return pl.pallas_call(
        _comb0_body,
        grid=(grid,),
        in_specs=[
            pl.BlockSpec((block_rows, ROW), lambda i: (i, 0)),
            pl.BlockSpec((block_rows, FEAT), lambda i: (i, 0)),
            pl.BlockSpec((1, FEAT), lambda i: (0, 0)),
        ],
        out_specs=pl.BlockSpec((block_rows, FEAT), lambda i: (i, 0)),
        out_shape=jax.ShapeDtypeStruct((n, FEAT), jnp.float32),
    )(acc, s0, bias0.reshape(1, -1))


def _comb1_body(acc_ref, s1_ref, b_ref, out_ref):
    a = acc_ref[...]
    tot = None
    for h in range(HEADS):
        m = a[:, h * HID:(h + 1) * HID]
        d = a[:, FEAT + h:FEAT + h + 1]
        v = m / (d + 1e-16)
        tot = v if tot is None else tot + v
    out_ref[...] = tot * (1.0 / HEADS) + b_ref[...] + s1_ref[...]


def _comb1(acc, s1, bias1):
    n = s1.shape[0]
    return pl.pallas_call(
        _comb1_body,
        in_specs=[
            pl.BlockSpec((n, ROW), lambda: (0, 0)),
            pl.BlockSpec((n, HID), lambda: (0, 0)),
            pl.BlockSpec((1, HID), lambda: (0, 0)),
        ],
        out_specs=pl.BlockSpec((n, HID), lambda: (0, 0)),
        out_shape=jax.ShapeDtypeStruct((n, HID), jnp.float32),
    )(acc, s1, bias1.reshape(1, -1))


def kernel(x, edge_index0, edge_index1, W0, b0, att0, bias0,
           W1, b1, att1, bias1, SW0, Sb0, SW1, Sb1):
    ei0 = edge_index0.astype(jnp.int32)
    ei1 = edge_index1.astype(jnp.int32)
    xt = x[:N1]
    z0, s0 = _proj(xt, W0, b0, SW0, Sb0, block_rows=1000)
    pad = jnp.arange(E0P - E0, dtype=jnp.int32)
    src0 = jnp.concatenate([ei0[0], pad % N1])
    dst0 = jnp.concatenate([ei0[1], N1 + pad % (NDP0 - N1)])
    acc0 = _edge0(z0, src0, dst0, att0.reshape(-1))
    h = _comb0(acc0.reshape(NDP0, ROW)[:N1], s0, bias0, block_rows=1000)
    ht = h[:N2]
    z1, s1 = _proj(ht, W1, b1, SW1, Sb1, block_rows=N2)
    acc1 = _edge1(z1, ei1[0], ei1[1], att1.reshape(-1))
    return _comb1(acc1.reshape(N2, ROW), s1, bias1)


# speedup vs baseline: 1.1196x; 1.1196x over previous
import functools

import jax
import jax.numpy as jnp
from jax import lax
from jax.experimental import pallas as pl
from jax.experimental.pallas import tpu as pltpu
from jax.experimental.pallas import tpu_sc as plsc

N0, N1, N2 = 50000, 10000, 2048
HEADS, HID = 4, 32
FEAT = HEADS * HID
ROW = 144
NEG_SLOPE = 0.2
W = 128
NS = 16
NC = 2


def _proj_body(x_ref, W_ref, b_ref, SW_ref, Sb_ref, z_ref, s_ref):
    xb = x_ref[...]
    z_ref[...] = (
        jnp.dot(xb, W_ref[...], preferred_element_type=jnp.float32) + b_ref[...]
    )
    s_ref[...] = (
        jnp.dot(xb, SW_ref[...], preferred_element_type=jnp.float32) + Sb_ref[...]
    )


def _proj(x, Wm, b, SW, Sb, block_rows):
    n, k = x.shape
    m1 = Wm.shape[1]
    m2 = SW.shape[1]
    grid = n // block_rows
    return pl.pallas_call(
        _proj_body,
        grid=(grid,),
        in_specs=[
            pl.BlockSpec((block_rows, k), lambda i: (i, 0)),
            pl.BlockSpec((k, m1), lambda i: (0, 0)),
            pl.BlockSpec((1, m1), lambda i: (0, 0)),
            pl.BlockSpec((k, m2), lambda i: (0, 0)),
            pl.BlockSpec((1, m2), lambda i: (0, 0)),
        ],
        out_specs=[
            pl.BlockSpec((block_rows, m1), lambda i: (i, 0)),
            pl.BlockSpec((block_rows, m2), lambda i: (i, 0)),
        ],
        out_shape=[
            jax.ShapeDtypeStruct((n, m1), jnp.float32),
            jax.ShapeDtypeStruct((n, m2), jnp.float32),
        ],
    )(x, Wm, b.reshape(1, -1), SW, Sb.reshape(1, -1))


def _chunks(total, maxc):
    out = []
    off = 0
    while off < total:
        c = min(maxc, total - off)
        out.append((off, c))
        off += c
    return out


def _make_edge_kernel(E, ND, ZR, R):
    ECR = E // NS
    NWIN = ECR // W
    assert ECR % W == 0 and ND % (NC * R * 128) == 0
    QND = ND // (NC * R)
    NDL = QND + 128
    RPTZ = NDL // NS
    RPTO = QND // NS
    mesh = plsc.VectorSubcoreMesh(core_axis_name="c", subcore_axis_name="s")

    RW = 512
    T = ECR // (2 * RW)
    PCAP = 384
    SCW = ECR + 2 * W
    assert ECR % (2 * RW) == 0

    def _body(z_hbm, src_hbm, dst_hbm, att_hbm, out_hbm, scr_hbm,
              raws, rawd, pends, pendg, pendd, procs, procg, procd,
              zsrc, zdst, upd, attv, acc,
              sem1, sem2, sas, sad, sbs, sbd):
        cid = lax.axis_index("c")
        sid = lax.axis_index("s")
        wid = cid * NS + sid
        lanes = lax.broadcasted_iota(jnp.int32, (16,), 0)
        zero16 = jnp.zeros((16,), jnp.float32)

        pltpu.sync_copy(att_hbm, attv)

        atts = [attv[pl.ds(j * 16, 16)] for j in range(8)]

        def ebody(kk, pb):
            for u in range(4):
                k = kk * 4 + u
                zs_list = []
                exs = []
                for h in range(HEADS):
                    part = None
                    for j in (2 * h, 2 * h + 1):
                        zs = zsrc[pb + k, pl.ds(j * 16, 16)]
                        zd = zdst[pb + k, pl.ds(j * 16, 16)]
                        zs_list.append(zs)
                        e = zs + zd
                        m = jnp.maximum(e, NEG_SLOPE * e) * atts[j]
                        part = m if part is None else part + m
                    s = jnp.sum(part)
                    exs.append(jnp.exp(lax.broadcast(s, (16,))))
                for h in range(HEADS):
                    for jj in range(2):
                        j = 2 * h + jj
                        upd[k, pl.ds(j * 16, 16)] = zs_list[j] * exs[h]
                tail = jnp.where(
                    lanes == 0, exs[0],
                    jnp.where(lanes == 1, exs[1],
                              jnp.where(lanes == 2, exs[2],
                                        jnp.where(lanes == 3, exs[3],
                                                  zero16))))
                upd[k, pl.ds(FEAT, 16)] = tail
            return pb

        def dump_window(nw):
            pltpu.sync_copy(pends.at[pl.ds(0, W)],
                            scr_hbm.at[wid, 0, pl.ds(nw * W, W)])
            pltpu.sync_copy(pendg.at[pl.ds(0, W)],
                            scr_hbm.at[wid, 1, pl.ds(nw * W, W)])
            pltpu.sync_copy(pendd.at[pl.ds(0, W)],
                            scr_hbm.at[wid, 2, pl.ds(nw * W, W)])
            for j in range((PCAP - W) // 16):
                pends[pl.ds(j * 16, 16)] = pends[pl.ds(W + j * 16, 16)]
                pendg[pl.ds(j * 16, 16)] = pendg[pl.ds(W + j * 16, 16)]
                pendd[pl.ds(j * 16, 16)] = pendd[pl.ds(W + j * 16, 16)]

        t0 = sid * ECR

        def issue_raw(w, rb, ss, sd):
            c1 = pltpu.async_copy(src_hbm.at[pl.ds(t0 + w * RW, RW)],
                                  raws.at[rb], ss)
            c2 = pltpu.async_copy(dst_hbm.at[pl.ds(t0 + w * RW, RW)],
                                  rawd.at[rb], sd)
            del c1, c2

        def wait_raw(rb, ss, sd):
            pltpu.make_async_copy(src_hbm.at[pl.ds(0, RW)],
                                  raws.at[rb], ss).wait()
            pltpu.make_async_copy(dst_hbm.at[pl.ds(0, RW)],
                                  rawd.at[rb], sd).wait()

        def rbody(r, _rc):
            lo = (cid * R + r) * QND
            lov = lax.broadcast(lo, (16,))
            hiv = lax.broadcast(lo + QND, (16,))

            def zbody(rr, c):
                for u in range(4):
                    for j in range(ROW // 16):
                        upd[rr * 4 + u, pl.ds(j * 16, 16)] = zero16
                return c

            lax.fori_loop(0, W // 4, zbody, 0)
            zbase = sid * RPTZ
            for off, c in _chunks(RPTZ, W):
                pltpu.sync_copy(upd.at[pl.ds(0, c)],
                                acc.at[pl.ds(zbase + off, c)])
            plsc.subcore_barrier()

            def compact_half(rb, g0, state):
                ptr, nw = state
                for j in range(g0, g0 + W // 16):
                    s = raws[rb, pl.ds(j * 16, 16)]
                    d = rawd[rb, pl.ds(j * 16, 16)]
                    msk = (d >= lov) & (d < hiv)
                    mi = msk.astype(jnp.int32)
                    pos = plsc.cumsum(mi) + lax.broadcast(ptr, (16,)) - 1
                    plsc.store_scatter(pends, [pos], s, mask=msk)
                    plsc.store_scatter(pendg, [pos], d, mask=msk)
                    plsc.store_scatter(pendd, [pos], d - lov, mask=msk)
                    ptr = ptr + jnp.sum(mi)

                @pl.when(ptr >= W)
                def _():
                    dump_window(nw)

                full = ptr >= W
                return (jnp.where(full, ptr - W, ptr),
                        jnp.where(full, nw + 1, nw))

            issue_raw(0, 0, sas, sad)

            def tbody(t, state):
                issue_raw(2 * t + 1, 1, sbs, sbd)
                wait_raw(0, sas, sad)
                for g0 in range(0, RW // 16, W // 16):
                    state = compact_half(0, g0, state)

                @pl.when(t + 1 < T)
                def _():
                    issue_raw(2 * t + 2, 0, sas, sad)

                wait_raw(1, sbs, sbd)
                for g0 in range(0, RW // 16, W // 16):
                    state = compact_half(1, g0, state)
                return state

            ptr, nw = lax.fori_loop(0, T, tbody,
                                    (jnp.int32(0), jnp.int32(0)))

            for j in range(PCAP // 16):
                gi = lanes + j * 16
                pad = gi >= lax.broadcast(ptr, (16,))
                tsrc = gi % ZR
                tdst = QND + (gi % 128)
                sv = pends[pl.ds(j * 16, 16)]
                pends[pl.ds(j * 16, 16)] = jnp.where(pad, tsrc, sv)
                gv = pendg[pl.ds(j * 16, 16)]
                pendg[pl.ds(j * 16, 16)] = jnp.where(pad, tsrc, gv)
                dv = pendd[pl.ds(j * 16, 16)]
                pendd[pl.ds(j * 16, 16)] = jnp.where(pad, tdst, dv)
            dump_window(nw)
            dump_window(nw + 1)
            nw = nw + 2

            def fetch(w, q):
                pltpu.sync_copy(scr_hbm.at[wid, 0, pl.ds(w * W, W)],
                                procs.at[q])
                pltpu.sync_copy(scr_hbm.at[wid, 1, pl.ds(w * W, W)],
                                procg.at[q])
                pltpu.sync_copy(scr_hbm.at[wid, 2, pl.ds(w * W, W)],
                                procd.at[q])
                c1 = pltpu.async_copy(z_hbm.at[procs.at[q]],
                                      zsrc.at[pl.ds(q * W, W)], sem1)
                c2 = pltpu.async_copy(z_hbm.at[procg.at[q]],
                                      zdst.at[pl.ds(q * W, W)], sem2)
                del c1, c2

            fetch(jnp.int32(0), jnp.int32(0))

            def pbody(w, c):
                p = w % 2

                @pl.when(w + 1 < nw)
                def _():
                    fetch(w + 1, (w + 1) % 2)

                pltpu.make_async_copy(z_hbm.at[pl.ds(0, W)],
                                      zsrc.at[pl.ds(p * W, W)], sem1).wait()
                pltpu.make_async_copy(z_hbm.at[pl.ds(0, W)],
                                      zdst.at[pl.ds(p * W, W)], sem2).wait()
                lax.fori_loop(0, W // 4, ebody, p * W)
                pltpu.sync_copy(upd, acc.at[procd.at[p]], add=True)
                return c

            lax.fori_loop(0, nw, pbody, 0)

            plsc.subcore_barrier()
            obase = sid * RPTO
            for off, c in _chunks(RPTO, W):
                pltpu.sync_copy(acc.at[pl.ds(obase + off, c)],
                                out_hbm.at[cid * R + r, pl.ds(obase + off, c)])
            plsc.subcore_barrier()
            return _rc

        lax.fori_loop(0, R, rbody, 0)

    @functools.partial(
        pl.kernel,
        mesh=mesh,
        compiler_params=pltpu.CompilerParams(
            needs_layout_passes=False, use_tc_tiling_on_sc=False),
        out_type=(
            jax.ShapeDtypeStruct((NC * R, QND, ROW), jnp.float32),
            jax.ShapeDtypeStruct((NC * NS, 3, SCW), jnp.int32),
        ),
        scratch_types=[
            pltpu.VMEM((2, RW), jnp.int32),
            pltpu.VMEM((2, RW), jnp.int32),
            pltpu.VMEM((PCAP,), jnp.int32),
            pltpu.VMEM((PCAP,), jnp.int32),
            pltpu.VMEM((PCAP,), jnp.int32),
            pltpu.VMEM((2, W), jnp.int32),
            pltpu.VMEM((2, W), jnp.int32),
            pltpu.VMEM((2, W), jnp.int32),
            pltpu.VMEM((2 * W, FEAT), jnp.float32),
            pltpu.VMEM((2 * W, FEAT), jnp.float32),
            pltpu.VMEM((W, ROW), jnp.float32),
            pltpu.VMEM((FEAT,), jnp.float32),
            pltpu.VMEM_SHARED((NDL, ROW), jnp.float32),
            pltpu.SemaphoreType.DMA,
            pltpu.SemaphoreType.DMA,
            pltpu.SemaphoreType.DMA,
            pltpu.SemaphoreType.DMA,
            pltpu.SemaphoreType.DMA,
            pltpu.SemaphoreType.DMA,
        ],
    )
    def edge_kernel(z_hbm, src_hbm, dst_hbm, att_hbm, out_hbm, scr_hbm,
                    *scratch):
        _body(z_hbm, src_hbm, dst_hbm, att_hbm, out_hbm, scr_hbm, *scratch)

    def run(z, src, dst, att):
        return edge_kernel(z, src, dst, att)[0]

    return run


E0, E0P = 320000, 327680
E1 = 65536
NDP0 = 10240
_edge0 = _make_edge_kernel(E0P, NDP0, N1, 4)
_edge1 = _make_edge_kernel(E1, N2, N2, 2)


def _comb0_body(acc_ref, s0_ref, b_ref, h_ref):
    a = acc_ref[...]
    parts = []
    for h in range(HEADS):
        m = a[:, h * HID:(h + 1) * HID]
        d = a[:, FEAT + h:FEAT + h + 1]
        parts.append(m / (d + 1e-16))
    o = jnp.concatenate(parts, axis=1) + b_ref[...] + s0_ref[...]
    h_ref[...] = jnp.where(o > 0, o, jnp.exp(jnp.minimum(o, 0.0)) - 1.0)


def _comb0(acc, s0, bias0, block_rows):
    n = s0.shape[0]
    grid = n // block_rows
    return pl.pallas_call(
        _comb0_body,
        grid=(grid,),
        in_specs=[
            pl.BlockSpec((block_rows, ROW), lambda i: (i, 0)),
            pl.BlockSpec((block_rows, FEAT), lambda i: (i, 0)),
            pl.BlockSpec((1, FEAT), lambda i: (0, 0)),
        ],
        out_specs=pl.BlockSpec((block_rows, FEAT), lambda i: (i, 0)),
        out_shape=jax.ShapeDtypeStruct((n, FEAT), jnp.float32),
    )(acc, s0, bias0.reshape(1, -1))


def _comb1_body(acc_ref, s1_ref, b_ref, out_ref):
    a = acc_ref[...]
    tot = None
    for h in range(HEADS):
        m = a[:, h * HID:(h + 1) * HID]
        d = a[:, FEAT + h:FEAT + h + 1]
        v = m / (d + 1e-16)
        tot = v if tot is None else tot + v
    out_ref[...] = tot * (1.0 / HEADS) + b_ref[...] + s1_ref[...]


def _comb1(acc, s1, bias1):
    n = s1.shape[0]
    return pl.pallas_call(
        _comb1_body,
        in_specs=[
            pl.BlockSpec((n, ROW), lambda: (0, 0)),
            pl.BlockSpec((n, HID), lambda: (0, 0)),
            pl.BlockSpec((1, HID), lambda: (0, 0)),
        ],
        out_specs=pl.BlockSpec((n, HID), lambda: (0, 0)),
        out_shape=jax.ShapeDtypeStruct((n, HID), jnp.float32),
    )(acc, s1, bias1.reshape(1, -1))


def kernel(x, edge_index0, edge_index1, W0, b0, att0, bias0,
           W1, b1, att1, bias1, SW0, Sb0, SW1, Sb1):
    ei0 = edge_index0.astype(jnp.int32)
    ei1 = edge_index1.astype(jnp.int32)
    xt = x[:N1]
    z0, s0 = _proj(xt, W0, b0, SW0, Sb0, block_rows=1000)
    pad = jnp.arange(E0P - E0, dtype=jnp.int32)
    src0 = jnp.concatenate([ei0[0], pad % N1])
    dst0 = jnp.concatenate([ei0[1], N1 + pad % (NDP0 - N1)])
    acc0 = _edge0(z0, src0, dst0, att0.reshape(-1))
    h = _comb0(acc0.reshape(NDP0, ROW)[:N1], s0, bias0, block_rows=1000)
    ht = h[:N2]
    z1, s1 = _proj(ht, W1, b1, SW1, Sb1, block_rows=N2)
    acc1 = _edge1(z1, ei1[0], ei1[1], att1.reshape(-1))
    return _comb1(acc1.reshape(N2, ROW), s1, bias1)


# speedup vs baseline: 1.2313x; 1.0998x over previous
import functools

import jax
import jax.numpy as jnp
from jax import lax
from jax.experimental import pallas as pl
from jax.experimental.pallas import tpu as pltpu
from jax.experimental.pallas import tpu_sc as plsc

N0, N1, N2 = 50000, 10000, 2048
HEADS, HID = 4, 32
FEAT = HEADS * HID
ROW = 144
NEG_SLOPE = 0.2
W = 128
NS = 16
NC = 2


def _proj_body(x_ref, W_ref, b_ref, SW_ref, Sb_ref, z_ref, s_ref):
    xb = x_ref[...]
    z_ref[...] = (
        jnp.dot(xb, W_ref[...], preferred_element_type=jnp.float32) + b_ref[...]
    )
    s_ref[...] = (
        jnp.dot(xb, SW_ref[...], preferred_element_type=jnp.float32) + Sb_ref[...]
    )


def _proj(x, Wm, b, SW, Sb, block_rows):
    n, k = x.shape
    m1 = Wm.shape[1]
    m2 = SW.shape[1]
    grid = n // block_rows
    return pl.pallas_call(
        _proj_body,
        grid=(grid,),
        in_specs=[
            pl.BlockSpec((block_rows, k), lambda i: (i, 0)),
            pl.BlockSpec((k, m1), lambda i: (0, 0)),
            pl.BlockSpec((1, m1), lambda i: (0, 0)),
            pl.BlockSpec((k, m2), lambda i: (0, 0)),
            pl.BlockSpec((1, m2), lambda i: (0, 0)),
        ],
        out_specs=[
            pl.BlockSpec((block_rows, m1), lambda i: (i, 0)),
            pl.BlockSpec((block_rows, m2), lambda i: (i, 0)),
        ],
        out_shape=[
            jax.ShapeDtypeStruct((n, m1), jnp.float32),
            jax.ShapeDtypeStruct((n, m2), jnp.float32),
        ],
    )(x, Wm, b.reshape(1, -1), SW, Sb.reshape(1, -1))


def _chunks(total, maxc):
    out = []
    off = 0
    while off < total:
        c = min(maxc, total - off)
        out.append((off, c))
        off += c
    return out


def _make_edge_kernel(E, ND, ZR, R):
    ECR = E // NS
    NWIN = ECR // W
    assert ECR % W == 0 and ND % (NC * R * 128) == 0
    QND = ND // (NC * R)
    NDL = QND + 8
    RPTO = QND // NS
    mesh = plsc.VectorSubcoreMesh(core_axis_name="c", subcore_axis_name="s")

    RW = 512
    T = ECR // (2 * RW)
    PCAP = 384
    SCW = ECR + 2 * W
    assert ECR % (2 * RW) == 0

    def _body(z_hbm, src_hbm, dst_hbm, att_hbm, out_hbm, scr_hbm,
              raws, rawd, pends, pendg, pendd, procs, procg, procd,
              zsrc, zdst, upd, attv, acc,
              sem1, sem2, sas, sad, sbs, sbd):
        cid = lax.axis_index("c")
        sid = lax.axis_index("s")
        wid = cid * NS + sid
        lanes = lax.broadcasted_iota(jnp.int32, (16,), 0)
        zero16 = jnp.zeros((16,), jnp.float32)

        pltpu.sync_copy(att_hbm, attv)

        atts = [attv[pl.ds(j * 16, 16)] for j in range(8)]

        def ebody(kk, pb):
            for u in range(4):
                k = kk * 4 + u
                zs_list = []
                exs = []
                for h in range(HEADS):
                    part = None
                    for j in (2 * h, 2 * h + 1):
                        zs = zsrc[pb + k, pl.ds(j * 16, 16)]
                        zd = zdst[pb + k, pl.ds(j * 16, 16)]
                        zs_list.append(zs)
                        e = zs + zd
                        m = jnp.maximum(e, NEG_SLOPE * e) * atts[j]
                        part = m if part is None else part + m
                    s = jnp.sum(part)
                    exs.append(jnp.exp(lax.broadcast(s, (16,))))
                for h in range(HEADS):
                    for jj in range(2):
                        j = 2 * h + jj
                        upd[k, pl.ds(j * 16, 16)] = zs_list[j] * exs[h]
                tail = jnp.where(
                    lanes == 0, exs[0],
                    jnp.where(lanes == 1, exs[1],
                              jnp.where(lanes == 2, exs[2],
                                        jnp.where(lanes == 3, exs[3],
                                                  zero16))))
                upd[k, pl.ds(FEAT, 16)] = tail
            return pb

        def dump_window(nw):
            pltpu.sync_copy(pends.at[pl.ds(0, W)],
                            scr_hbm.at[wid, 0, pl.ds(nw * W, W)])
            pltpu.sync_copy(pendg.at[pl.ds(0, W)],
                            scr_hbm.at[wid, 1, pl.ds(nw * W, W)])
            pltpu.sync_copy(pendd.at[pl.ds(0, W)],
                            scr_hbm.at[wid, 2, pl.ds(nw * W, W)])
            for j in range((PCAP - W) // 16):
                pends[pl.ds(j * 16, 16)] = pends[pl.ds(W + j * 16, 16)]
                pendg[pl.ds(j * 16, 16)] = pendg[pl.ds(W + j * 16, 16)]
                pendd[pl.ds(j * 16, 16)] = pendd[pl.ds(W + j * 16, 16)]

        t0 = sid * ECR

        def issue_raw(w, rb, ss, sd):
            c1 = pltpu.async_copy(src_hbm.at[pl.ds(t0 + w * RW, RW)],
                                  raws.at[rb], ss)
            c2 = pltpu.async_copy(dst_hbm.at[pl.ds(t0 + w * RW, RW)],
                                  rawd.at[rb], sd)
            del c1, c2

        def wait_raw(rb, ss, sd):
            pltpu.make_async_copy(src_hbm.at[pl.ds(0, RW)],
                                  raws.at[rb], ss).wait()
            pltpu.make_async_copy(dst_hbm.at[pl.ds(0, RW)],
                                  rawd.at[rb], sd).wait()

        def rbody(r, _rc):
            lo = (cid * R + r) * QND
            lov = lax.broadcast(lo, (16,))
            hiv = lax.broadcast(lo + QND, (16,))

            def zbody(rr, c):
                for u in range(4):
                    for j in range(ROW // 16):
                        upd[rr * 4 + u, pl.ds(j * 16, 16)] = zero16
                return c

            lax.fori_loop(0, W // 4, zbody, 0)
            zbase = sid * RPTO
            for off, c in _chunks(RPTO, W):
                pltpu.sync_copy(upd.at[pl.ds(0, c)],
                                acc.at[pl.ds(zbase + off, c)])

            @pl.when(sid == 0)
            def _():
                pltpu.sync_copy(upd.at[pl.ds(0, 8)],
                                acc.at[pl.ds(QND, 8)])

            plsc.subcore_barrier()

            def compact_half(rb, g0, state):
                ptr, nw = state
                for j in range(g0, g0 + W // 16):
                    s = raws[rb, pl.ds(j * 16, 16)]
                    d = rawd[rb, pl.ds(j * 16, 16)]
                    msk = (d >= lov) & (d < hiv)
                    mi = msk.astype(jnp.int32)
                    pos = plsc.cumsum(mi) + lax.broadcast(ptr, (16,)) - 1
                    plsc.store_scatter(pends, [pos], s, mask=msk)
                    plsc.store_scatter(pendg, [pos], d, mask=msk)
                    plsc.store_scatter(pendd, [pos], d - lov, mask=msk)
                    ptr = ptr + jnp.sum(mi)

                @pl.when(ptr >= W)
                def _():
                    dump_window(nw)

                full = ptr >= W
                return (jnp.where(full, ptr - W, ptr),
                        jnp.where(full, nw + 1, nw))

            issue_raw(0, 0, sas, sad)

            def tbody(t, state):
                issue_raw(2 * t + 1, 1, sbs, sbd)
                wait_raw(0, sas, sad)
                for g0 in range(0, RW // 16, W // 16):
                    state = compact_half(0, g0, state)

                @pl.when(t + 1 < T)
                def _():
                    issue_raw(2 * t + 2, 0, sas, sad)

                wait_raw(1, sbs, sbd)
                for g0 in range(0, RW // 16, W // 16):
                    state = compact_half(1, g0, state)
                return state

            ptr, nw = lax.fori_loop(0, T, tbody,
                                    (jnp.int32(0), jnp.int32(0)))

            for j in range(PCAP // 16):
                gi = lanes + j * 16
                pad = gi >= lax.broadcast(ptr, (16,))
                tsrc = gi % ZR
                tdst = QND + (gi % 8)
                sv = pends[pl.ds(j * 16, 16)]
                pends[pl.ds(j * 16, 16)] = jnp.where(pad, tsrc, sv)
                gv = pendg[pl.ds(j * 16, 16)]
                pendg[pl.ds(j * 16, 16)] = jnp.where(pad, tsrc, gv)
                dv = pendd[pl.ds(j * 16, 16)]
                pendd[pl.ds(j * 16, 16)] = jnp.where(pad, tdst, dv)
            dump_window(nw)
            dump_window(nw + 1)
            nw = nw + 2

            def fetch(w, q):
                pltpu.sync_copy(scr_hbm.at[wid, 0, pl.ds(w * W, W)],
                                procs.at[q])
                pltpu.sync_copy(scr_hbm.at[wid, 1, pl.ds(w * W, W)],
                                procg.at[q])
                pltpu.sync_copy(scr_hbm.at[wid, 2, pl.ds(w * W, W)],
                                procd.at[q])
                c1 = pltpu.async_copy(z_hbm.at[procs.at[q]],
                                      zsrc.at[pl.ds(q * W, W)], sem1)
                c2 = pltpu.async_copy(z_hbm.at[procg.at[q]],
                                      zdst.at[pl.ds(q * W, W)], sem2)
                del c1, c2

            fetch(jnp.int32(0), jnp.int32(0))

            def pbody(w, c):
                p = w % 2

                @pl.when(w + 1 < nw)
                def _():
                    fetch(w + 1, (w + 1) % 2)

                pltpu.make_async_copy(z_hbm.at[pl.ds(0, W)],
                                      zsrc.at[pl.ds(p * W, W)], sem1).wait()
                pltpu.make_async_copy(z_hbm.at[pl.ds(0, W)],
                                      zdst.at[pl.ds(p * W, W)], sem2).wait()
                lax.fori_loop(0, W // 4, ebody, p * W)
                pltpu.sync_copy(upd, acc.at[procd.at[p]], add=True)
                return c

            lax.fori_loop(0, nw, pbody, 0)

            plsc.subcore_barrier()
            obase = sid * RPTO
            for off, c in _chunks(RPTO, W):
                pltpu.sync_copy(acc.at[pl.ds(obase + off, c)],
                                out_hbm.at[cid * R + r, pl.ds(obase + off, c)])
            plsc.subcore_barrier()
            return _rc

        lax.fori_loop(0, R, rbody, 0)

    @functools.partial(
        pl.kernel,
        mesh=mesh,
        compiler_params=pltpu.CompilerParams(
            needs_layout_passes=False, use_tc_tiling_on_sc=False),
        out_type=(
            jax.ShapeDtypeStruct((NC * R, QND, ROW), jnp.float32),
            jax.ShapeDtypeStruct((NC * NS, 3, SCW), jnp.int32),
        ),
        scratch_types=[
            pltpu.VMEM((2, RW), jnp.int32),
            pltpu.VMEM((2, RW), jnp.int32),
            pltpu.VMEM((PCAP,), jnp.int32),
            pltpu.VMEM((PCAP,), jnp.int32),
            pltpu.VMEM((PCAP,), jnp.int32),
            pltpu.VMEM((2, W), jnp.int32),
            pltpu.VMEM((2, W), jnp.int32),
            pltpu.VMEM((2, W), jnp.int32),
            pltpu.VMEM((2 * W, FEAT), jnp.float32),
            pltpu.VMEM((2 * W, FEAT), jnp.float32),
            pltpu.VMEM((W, ROW), jnp.float32),
            pltpu.VMEM((FEAT,), jnp.float32),
            pltpu.VMEM_SHARED((NDL, ROW), jnp.float32),
            pltpu.SemaphoreType.DMA,
            pltpu.SemaphoreType.DMA,
            pltpu.SemaphoreType.DMA,
            pltpu.SemaphoreType.DMA,
            pltpu.SemaphoreType.DMA,
            pltpu.SemaphoreType.DMA,
        ],
    )
    def edge_kernel(z_hbm, src_hbm, dst_hbm, att_hbm, out_hbm, scr_hbm,
                    *scratch):
        _body(z_hbm, src_hbm, dst_hbm, att_hbm, out_hbm, scr_hbm, *scratch)

    def run(z, src, dst, att):
        return edge_kernel(z, src, dst, att)[0]

    return run


E0, E0P = 320000, 327680
E1 = 65536
NDP0 = 10240
_edge0 = _make_edge_kernel(E0P, NDP0, N1, 2)
_edge1 = _make_edge_kernel(E1, N2, N2, 4)


def _comb0_body(acc_ref, s0_ref, b_ref, h_ref):
    a = acc_ref[...]
    parts = []
    for h in range(HEADS):
        m = a[:, h * HID:(h + 1) * HID]
        d = a[:, FEAT + h:FEAT + h + 1]
        parts.append(m / (d + 1e-16))
    o = jnp.concatenate(parts, axis=1) + b_ref[...] + s0_ref[...]
    h_ref[...] = jnp.where(o > 0, o, jnp.exp(jnp.minimum(o, 0.0)) - 1.0)


def _comb0(acc, s0, bias0, block_rows):
    n = s0.shape[0]
    grid = n // block_rows
    return pl.pallas_call(
        _comb0_body,
        grid=(grid,),
        in_specs=[
            pl.BlockSpec((block_rows, ROW), lambda i: (i, 0)),
            pl.BlockSpec((block_rows, FEAT), lambda i: (i, 0)),
            pl.BlockSpec((1, FEAT), lambda i: (0, 0)),
        ],
        out_specs=pl.BlockSpec((block_rows, FEAT), lambda i: (i, 0)),
        out_shape=jax.ShapeDtypeStruct((n, FEAT), jnp.float32),
    )(acc, s0, bias0.reshape(1, -1))


def _comb1_body(acc_ref, s1_ref, b_ref, out_ref):
    a = acc_ref[...]
    tot = None
    for h in range(HEADS):
        m = a[:, h * HID:(h + 1) * HID]
        d = a[:, FEAT + h:FEAT + h + 1]
        v = m / (d + 1e-16)
        tot = v if tot is None else tot + v
    out_ref[...] = tot * (1.0 / HEADS) + b_ref[...] + s1_ref[...]


def _comb1(acc, s1, bias1):
    n = s1.shape[0]
    return pl.pallas_call(
        _comb1_body,
        in_specs=[
            pl.BlockSpec((n, ROW), lambda: (0, 0)),
            pl.BlockSpec((n, HID), lambda: (0, 0)),
            pl.BlockSpec((1, HID), lambda: (0, 0)),
        ],
        out_specs=pl.BlockSpec((n, HID), lambda: (0, 0)),
        out_shape=jax.ShapeDtypeStruct((n, HID), jnp.float32),
    )(acc, s1, bias1.reshape(1, -1))


def kernel(x, edge_index0, edge_index1, W0, b0, att0, bias0,
           W1, b1, att1, bias1, SW0, Sb0, SW1, Sb1):
    ei0 = edge_index0.astype(jnp.int32)
    ei1 = edge_index1.astype(jnp.int32)
    xt = x[:N1]
    z0, s0 = _proj(xt, W0, b0, SW0, Sb0, block_rows=1000)
    pad = jnp.arange(E0P - E0, dtype=jnp.int32)
    src0 = jnp.concatenate([ei0[0], pad % N1])
    dst0 = jnp.concatenate([ei0[1], N1 + pad % (NDP0 - N1)])
    acc0 = _edge0(z0, src0, dst0, att0.reshape(-1))
    h = _comb0(acc0.reshape(NDP0, ROW)[:N1], s0, bias0, block_rows=1000)
    ht = h[:N2]
    z1, s1 = _proj(ht, W1, b1, SW1, Sb1, block_rows=N2)
    acc1 = _edge1(z1, ei1[0], ei1[1], att1.reshape(-1))
    return _comb1(acc1.reshape(N2, ROW), s1, bias1)
